# baseline (speedup 1.0000x reference)
import functools

import jax
import jax.numpy as jnp
from jax.experimental import pallas as pl
from jax.experimental.pallas import tpu as pltpu

_NEG_INF = -1e30
_KERNEL_WINS = (3, 4, 5)
_DIM_CHANNEL = 256
_PAD_ID = 0
_NUM_CLASS = 20


def _round_up(x, m):
    return ((x + m - 1) // m) * m


def _fused_kernel(xr_ref, pk_ref, mpad_ref, tbl_ref, wtap_ref,
                  w1_ref, b1_ref, w2_ref, b2_ref, out_ref, gba, gbb, ebuf,
                  *, kernel_wins, dim_channel, bt, l_seq, l_ext):
    Bt, L, L_ext = bt, l_seq, l_ext
    E = tbl_ref.shape[2]
    M = Bt * L_ext
    C = dim_channel

    G = next(g for g in (40, 32, 16, 8, 4, 2, 1) if M % g == 0)
    GB = G // 8
    stair = ((jax.lax.broadcasted_iota(jnp.int32, (8, 128), 1) >> 4) ==
             jax.lax.broadcasted_iota(jnp.int32, (8, 128), 0)
             ).astype(jnp.bfloat16)
    zed = jnp.zeros((8, 128), jnp.bfloat16)
    bufs = (gba, gbb)
    for g in range(M // G):
        bb = bufs[g & 1]
        for i in range(G):
            bb[i] = tbl_ref[xr_ref[0, 0, g * G + i]]
        rowblocks = []
        for c in range(GB):
            pkrow = pk_ref[0, GB * g + c:GB * g + c + 1, :].reshape(1, 128)
            tile = jnp.broadcast_to(pkrow, (8, 128)) * stair
            rowblocks.append(jnp.concatenate(
                [zed] * c + [tile] + [zed] * (GB - 1 - c), axis=1))
        sel = jnp.concatenate(rowblocks, axis=0)
        st = bb[...].reshape(16 * G, E)
        e_g = jnp.dot(sel, st, preferred_element_type=jnp.float32)
        ebuf[G * g:G * (g + 1), :] = e_g.astype(jnp.bfloat16)

    emb = ebuf[...].reshape(Bt, L_ext, E)

    n_taps = L_ext - L + 1
    unf = jnp.concatenate([emb[:, k:k + L, :] for k in range(n_taps)],
                          axis=-1).reshape(Bt * L, n_taps * E)

    mpad = mpad_ref[...]
    pooled = []
    for i, win in enumerate(kernel_wins):
        depth = win * E
        a = jnp.dot(unf[:, :depth], wtap_ref[:depth, i * C:(i + 1) * C],
                    preferred_element_type=jnp.float32)
        a = a.reshape(Bt, L, C)
        if win > 1:
            m = jnp.concatenate(
                [mpad[:, win - 1:],
                 jnp.full((Bt, win - 1), _NEG_INF, jnp.float32)], axis=1)
        else:
            m = mpad
        pooled.append(jnp.max(a + m[:, :, None], axis=1))
    pooled = jnp.concatenate(pooled, axis=-1)

    h = jnp.dot(pooled, w1_ref[...], preferred_element_type=jnp.float32)
    h = jnp.maximum(h + b1_ref[...], 0.0)
    out_ref[...] = jnp.dot(h, w2_ref[...],
                           preferred_element_type=jnp.float32) + b2_ref[...]


@jax.jit
def _forward(embed, wtap, w1, b1, w2, b2, x_ids):
    B, L = x_ids.shape
    E = embed.shape[1]
    KP, CP = wtap.shape
    n_taps = KP // E
    FP = w1.shape[1]
    NCP = w2.shape[1]
    L_ext = L + n_taps - 1

    Bt = 16
    B_pad = _round_up(B, Bt)
    NB = B_pad // Bt
    M = Bt * L_ext
    G2 = 2 if NB % 2 == 0 else 1
    grid = (G2, NB // G2)

    x_ext = jnp.pad(x_ids, ((0, B_pad - B), (0, n_taps - 1)),
                    constant_values=_PAD_ID)
    xf = x_ext.reshape(-1)
    xrow = (xf >> 4).reshape(NB, 1, M)
    pk = (
        (xf & 15).reshape(NB, M // 8, 8, 1) ==
        jnp.arange(16, dtype=jnp.int32).reshape(1, 1, 1, 16)
    ).astype(jnp.bfloat16).reshape(NB, M // 8, 128)
    V = embed.shape[0]
    tbl3 = embed.reshape(V // 16, 16, E)
    mpad = jnp.where(x_ext[:, :L] == _PAD_ID,
                     jnp.float32(-100.0), jnp.float32(0.0))

    Gs = next(g for g in (40, 32, 16, 8, 4, 2, 1) if M % g == 0)
    kern = functools.partial(_fused_kernel, kernel_wins=_KERNEL_WINS,
                             dim_channel=_DIM_CHANNEL, bt=Bt, l_seq=L,
                             l_ext=L_ext)
    nb2 = NB // G2
    out = pl.pallas_call(
        kern,
        out_shape=jax.ShapeDtypeStruct((B_pad, NCP), jnp.float32),
        grid=grid,
        in_specs=[
            pl.BlockSpec((1, 1, M), lambda i, j: (i * nb2 + j, 0, 0),
                         memory_space=pltpu.SMEM),
            pl.BlockSpec((1, M // 8, 128),
                         lambda i, j: (i * nb2 + j, 0, 0)),
            pl.BlockSpec((Bt, L), lambda i, j: (i * nb2 + j, 0)),
            pl.BlockSpec((V // 16, 16, E), lambda i, j: (0, 0, 0)),
            pl.BlockSpec((KP, CP), lambda i, j: (0, 0)),
            pl.BlockSpec((CP, FP), lambda i, j: (0, 0)),
            pl.BlockSpec((1, FP), lambda i, j: (0, 0)),
            pl.BlockSpec((FP, NCP), lambda i, j: (0, 0)),
            pl.BlockSpec((1, NCP), lambda i, j: (0, 0)),
        ],
        out_specs=pl.BlockSpec((Bt, NCP), lambda i, j: (i * nb2 + j, 0)),
        scratch_shapes=[pltpu.VMEM((Gs, 16, E), jnp.bfloat16),
                        pltpu.VMEM((Gs, 16, E), jnp.bfloat16),
                        pltpu.VMEM((M, E), jnp.bfloat16)],
        compiler_params=pltpu.CompilerParams(
            dimension_semantics=("parallel", "arbitrary"),
            vmem_limit_bytes=60 * 1024 * 1024),
    )(xrow, pk, mpad, tbl3, wtap, w1, b1, w2, b2)

    return out[:B, :_NUM_CLASS]


def kernel(embed, wtap, w1, b1, w2, b2, x_ids):
    return _forward(embed, wtap, w1, b1, w2, b2, x_ids)


# speedup vs baseline: 1.0288x; 1.0288x over previous
import functools

import jax
import jax.numpy as jnp
from jax.experimental import pallas as pl
from jax.experimental.pallas import tpu as pltpu

_NEG_INF = -1e30
_KERNEL_WINS = (3, 4, 5)
_DIM_CHANNEL = 256
_PAD_ID = 0
_NUM_CLASS = 20


def _round_up(x, m):
    return ((x + m - 1) // m) * m


def _fused_kernel(xr_ref, pk_ref, mpad_ref, tbl_ref, wtap_ref,
                  w1_ref, b1_ref, w2_ref, b2_ref, out_ref, gba, gbb, ebuf,
                  *, kernel_wins, dim_channel, bt, l_seq, l_ext):
    Bt, L, L_ext = bt, l_seq, l_ext
    E = tbl_ref.shape[2]
    M = Bt * L_ext
    C = dim_channel

    G = next(g for g in (40, 32, 16, 8, 4, 2, 1) if M % g == 0)
    GB = G // 8
    stair = ((jax.lax.broadcasted_iota(jnp.int32, (8, 128), 1) >> 4) ==
             jax.lax.broadcasted_iota(jnp.int32, (8, 128), 0)
             ).astype(jnp.bfloat16)
    zed = jnp.zeros((8, 128), jnp.bfloat16)
    bufs = (gba, gbb)
    for g in range(M // G):
        bb = bufs[g & 1]
        for i in range(G):
            bb[i] = tbl_ref[xr_ref[0, 0, g * G + i]]
        rowblocks = []
        for c in range(GB):
            pkrow = pk_ref[0, GB * g + c:GB * g + c + 1, :].reshape(1, 128)
            tile = jnp.broadcast_to(pkrow, (8, 128)) * stair
            rowblocks.append(jnp.concatenate(
                [zed] * c + [tile] + [zed] * (GB - 1 - c), axis=1))
        sel = jnp.concatenate(rowblocks, axis=0)
        st = bb[...].reshape(16 * G, E)
        e_g = jnp.dot(sel, st, preferred_element_type=jnp.float32)
        ebuf[G * g:G * (g + 1), :] = e_g.astype(jnp.bfloat16)

    emb = ebuf[...].reshape(Bt, L_ext, E)

    n_taps = L_ext - L + 1
    unf = jnp.concatenate([emb[:, k:k + L, :] for k in range(n_taps)],
                          axis=-1).reshape(Bt * L, n_taps * E)

    mpad = mpad_ref[...]
    pooled = []
    for i, win in enumerate(kernel_wins):
        depth = win * E
        a = jnp.dot(unf[:, :depth], wtap_ref[:depth, i * C:(i + 1) * C],
                    preferred_element_type=jnp.float32)
        a = a.reshape(Bt, L, C)
        if win > 1:
            m = jnp.concatenate(
                [mpad[:, win - 1:],
                 jnp.full((Bt, win - 1), _NEG_INF, jnp.float32)], axis=1)
        else:
            m = mpad
        pooled.append(jnp.max(a + m[:, :, None], axis=1))
    pooled = jnp.concatenate(pooled, axis=-1)

    h = jnp.dot(pooled, w1_ref[...], preferred_element_type=jnp.float32)
    h = jnp.maximum(h + b1_ref[...], 0.0)
    out_ref[...] = jnp.dot(h, w2_ref[...],
                           preferred_element_type=jnp.float32) + b2_ref[...]


@jax.jit
def _forward(embed, wtap, w1, b1, w2, b2, x_ids):
    B, L = x_ids.shape
    E = embed.shape[1]
    KP, CP = wtap.shape
    n_taps = KP // E
    FP = w1.shape[1]
    NCP = w2.shape[1]
    L_ext = L + n_taps - 1

    Bt = 32
    B_pad = _round_up(B, Bt)
    NB = B_pad // Bt
    M = Bt * L_ext
    G2 = 2 if NB % 2 == 0 else 1
    grid = (G2, NB // G2)

    x_ext = jnp.pad(x_ids, ((0, B_pad - B), (0, n_taps - 1)),
                    constant_values=_PAD_ID)
    xf = x_ext.reshape(-1)
    xrow = (xf >> 4).reshape(NB, 1, M)
    pk = (
        (xf & 15).reshape(NB, M // 8, 8, 1) ==
        jnp.arange(16, dtype=jnp.int32).reshape(1, 1, 1, 16)
    ).astype(jnp.bfloat16).reshape(NB, M // 8, 128)
    V = embed.shape[0]
    tbl3 = embed.reshape(V // 16, 16, E)
    mpad = jnp.where(x_ext[:, :L] == _PAD_ID,
                     jnp.float32(-100.0), jnp.float32(0.0))

    Gs = next(g for g in (40, 32, 16, 8, 4, 2, 1) if M % g == 0)
    kern = functools.partial(_fused_kernel, kernel_wins=_KERNEL_WINS,
                             dim_channel=_DIM_CHANNEL, bt=Bt, l_seq=L,
                             l_ext=L_ext)
    nb2 = NB // G2
    out = pl.pallas_call(
        kern,
        out_shape=jax.ShapeDtypeStruct((B_pad, NCP), jnp.float32),
        grid=grid,
        in_specs=[
            pl.BlockSpec((1, 1, M), lambda i, j: (i * nb2 + j, 0, 0),
                         memory_space=pltpu.SMEM),
            pl.BlockSpec((1, M // 8, 128),
                         lambda i, j: (i * nb2 + j, 0, 0)),
            pl.BlockSpec((Bt, L), lambda i, j: (i * nb2 + j, 0)),
            pl.BlockSpec((V // 16, 16, E), lambda i, j: (0, 0, 0)),
            pl.BlockSpec((KP, CP), lambda i, j: (0, 0)),
            pl.BlockSpec((CP, FP), lambda i, j: (0, 0)),
            pl.BlockSpec((1, FP), lambda i, j: (0, 0)),
            pl.BlockSpec((FP, NCP), lambda i, j: (0, 0)),
            pl.BlockSpec((1, NCP), lambda i, j: (0, 0)),
        ],
        out_specs=pl.BlockSpec((Bt, NCP), lambda i, j: (i * nb2 + j, 0)),
        scratch_shapes=[pltpu.VMEM((Gs, 16, E), jnp.bfloat16),
                        pltpu.VMEM((Gs, 16, E), jnp.bfloat16),
                        pltpu.VMEM((M, E), jnp.bfloat16)],
        compiler_params=pltpu.CompilerParams(
            dimension_semantics=("parallel", "arbitrary"),
            vmem_limit_bytes=60 * 1024 * 1024),
    )(xrow, pk, mpad, tbl3, wtap, w1, b1, w2, b2)

    return out[:B, :_NUM_CLASS]


def kernel(embed, wtap, w1, b1, w2, b2, x_ids):
    return _forward(embed, wtap, w1, b1, w2, b2, x_ids)
